# baseline (speedup 1.0000x reference)
import functools

import jax
import jax.numpy as jnp
from jax import lax
from jax.experimental import pallas as pl
from jax.experimental.pallas import tpu as pltpu
from jax.experimental.pallas import tpu_sc as plsc

_NC = 2
_NS = 16
_NW = _NC * _NS


def _multi_gather(pairs):
    specs = []
    ins = []
    scratch = []
    out_type = []
    for t, i in pairs:
        nrow = i.shape[0]
        d = t.shape[1]
        b_per_w = nrow // _NW
        cpw = min(128, b_per_w)
        nch = b_per_w // cpw
        wide = d % 128 == 0
        specs.append((d, b_per_w, cpw, nch, wide))
        scr = [
            pltpu.VMEM((nch, cpw), jnp.int32),
            pltpu.VMEM((cpw, d if wide else 128), t.dtype),
            pltpu.SemaphoreType.DMA,
        ]
        if wide:
            ins += [t, i.reshape(-1, cpw)]
            out_type.append(jax.ShapeDtypeStruct((nrow, d), t.dtype))
        else:
            assert d == 16
            ins += [t.reshape(-1, 128), i.reshape(-1, cpw)]
            out_type.append(
                jax.ShapeDtypeStruct((nrow * d // 128, 128), t.dtype))
            scr += [
                pltpu.VMEM((cpw,), jnp.int32),
                pltpu.VMEM((cpw * d // 128, 128), t.dtype),
            ]
        scratch.append(tuple(scr))

    n = len(specs)
    mesh = plsc.VectorSubcoreMesh(core_axis_name="c", subcore_axis_name="s")

    def body(*refs):
        tables = refs[0:2 * n:2]
        idxs = refs[1:2 * n:2]
        outs = refs[2 * n:3 * n]
        scr = refs[3 * n:]
        wid = lax.axis_index("s") * _NC + lax.axis_index("c")
        for g in range(n):
            d, b_per_w, cpw, nch, wide = specs[g]
            table, idx2d, out = tables[g], idxs[g], outs[g]
            idx_v = scr[g][0]
            pltpu.sync_copy(idx2d.at[pl.ds(wid * nch, nch)], idx_v)

            if wide:
                def chunk(c, _, table=table, out=out, idx_v=idx_v,
                          buf=scr[g][1], sem=scr[g][2],
                          base=wid * b_per_w, cpw=cpw):
                    pltpu.async_copy(table.at[idx_v.at[c]], buf, sem).wait()
                    pltpu.sync_copy(buf, out.at[pl.ds(base + c * cpw, cpw)])
                    return _
            else:
                def chunk(c, _, table=table, out=out, idx_v=idx_v,
                          buf=scr[g][1], sem=scr[g][2], idx8=scr[g][3],
                          obuf=scr[g][4], wrows=b_per_w * d // 128,
                          crows=cpw * d // 128, cpw=cpw):
                    lane = lax.iota(jnp.int32, 16)
                    for gg in range(cpw // 16):
                        tvec = idx_v[c, pl.ds(gg * 16, 16)]
                        idx8[pl.ds(gg * 16, 16)] = lax.shift_right_logical(
                            tvec, 3)
                    pltpu.async_copy(table.at[idx8], buf, sem).wait()
                    for gg in range(cpw // 16):
                        tvec = idx_v[c, pl.ds(gg * 16, 16)]
                        kbase = (tvec & 7) * 16
                        rows = lane + gg * 16
                        for j in range(16):
                            vals = plsc.load_gather(buf, [rows, kbase + j])
                            flat = rows * 16 + j
                            plsc.store_scatter(
                                obuf,
                                [lax.shift_right_logical(flat, 7), flat & 127],
                                vals)
                    pltpu.sync_copy(
                        obuf, out.at[pl.ds(wid * wrows + c * crows, crows)])
                    return _

            if nch == 1:
                chunk(0, None)
            else:
                lax.fori_loop(0, nch, chunk, None)

    f = pl.kernel(body, out_type=tuple(out_type), mesh=mesh,
                  scratch_types=tuple(scratch),
                  compiler_params=pltpu.CompilerParams(
                      needs_layout_passes=False))
    return f(*ins)


def _sc_fused_agg(ent_emb, e2_2d, r1_2d, r0_2d, scores_2d):
    mesh = plsc.VectorSubcoreMesh(core_axis_name="c", subcore_axis_name="s")
    CH = 128
    NCH = 64
    HALF = NCH // 2

    def body(ent, e2i, r1i, r0i, sco, agg_out, w0_out,
             idx_v, r1_v, r0_v, sco_v, stage, buf0, buf1, w0_v,
             sem0, sem1):
        wid = lax.axis_index("s") * _NC + lax.axis_index("c")
        lane = lax.iota(jnp.int32, 16)
        pltpu.sync_copy(e2i.at[pl.ds(wid * NCH, NCH)], idx_v)
        pltpu.sync_copy(r1i.at[pl.ds(wid * NCH, NCH)], r1_v)
        pltpu.sync_copy(r0i.at[pl.ds(wid * 4, 4)], r0_v)
        pltpu.sync_copy(sco.at[pl.ds(wid * 8, 8)], sco_v)

        def softmax16(svals):
            m = jnp.max(svals)
            es = jnp.exp(svals - m)
            return es / jnp.sum(es)

        def w0_item(i, _):
            rv = r0_v[i >> 3, pl.ds(pl.multiple_of((i & 7) * 16, 16), 16)]
            srow = jnp.broadcast_to(i >> 2, (16,))
            svals = plsc.load_gather(sco_v, [srow, (i & 3) * 32 + rv])
            w0_v[i >> 3, pl.ds(pl.multiple_of((i & 7) * 16, 16), 16)] = (
                softmax16(svals))
            return _
        lax.fori_loop(0, 32, w0_item, None)
        pltpu.sync_copy(w0_v, w0_out.at[pl.ds(wid * 4, 4)])

        bufs = (buf0, buf1)
        sems = (sem0, sem1)

        def issue(c, sub):
            pltpu.async_copy(ent.at[idx_v.at[jnp.minimum(c, NCH - 1)]],
                             bufs[sub], sems[sub])

        def process(c, cl, sub):
            buf = bufs[sub]
            item = c >> 1

            def group(g):
                coff = pl.multiple_of(g * 16, 16)
                rv = r1_v[c, pl.ds(coff, 16)]
                srow = jnp.broadcast_to(item >> 2, (16,))
                svals = plsc.load_gather(sco_v, [srow, (item & 3) * 32 + rv])
                w = softmax16(svals)
                srow16 = (cl >> 1) * 16 + (c & 1) * 8 + g
                wks = [jnp.broadcast_to(w[k], (16,)) for k in range(16)]
                for j in range(8):
                    parts = [
                        wks[k] * buf[g * 16 + k, pl.ds(j * 16, 16)]
                        for k in range(16)]
                    for step in (8, 4, 2, 1):
                        parts = [parts[t] + parts[t + step]
                                 for t in range(step)]
                    stage[srow16, pl.ds(j * 16, 16)] = parts[0]
            for g in range(8):
                group(g)

        issue(0, 0)
        issue(1, 1)
        for h in range(2):
            def pair(p, _, h=h):
                cl = 2 * p
                c = h * HALF + cl
                for sub in range(2):
                    pltpu.make_async_copy(ent.at[idx_v.at[0]], bufs[sub],
                                          sems[sub]).wait()
                    process(c + sub, cl + sub, sub)
                    issue(c + sub + 2, sub)
                return _
            lax.fori_loop(0, HALF // 2, pair, None)
            pltpu.sync_copy(
                stage, agg_out.at[pl.ds(wid * 512 + h * 256, 256)])
        pltpu.make_async_copy(ent.at[idx_v.at[0]], bufs[0], sems[0]).wait()
        pltpu.make_async_copy(ent.at[idx_v.at[0]], bufs[1], sems[1]).wait()

    f = pl.kernel(
        body,
        out_type=(jax.ShapeDtypeStruct((16384, 128), jnp.float32),
                  jax.ShapeDtypeStruct((128, 128), jnp.float32)),
        mesh=mesh,
        scratch_types=(
            pltpu.VMEM((NCH, CH), jnp.int32),
            pltpu.VMEM((NCH, CH), jnp.int32),
            pltpu.VMEM((4, 128), jnp.int32),
            pltpu.VMEM((8, 128), jnp.float32),
            pltpu.VMEM((256, 128), jnp.float32),
            pltpu.VMEM((CH, 128), jnp.float32),
            pltpu.VMEM((CH, 128), jnp.float32),
            pltpu.VMEM((4, 128), jnp.float32),
            pltpu.SemaphoreType.DMA,
            pltpu.SemaphoreType.DMA,
        ),
        compiler_params=pltpu.CompilerParams(needs_layout_passes=False))
    return f(ent_emb, e2_2d, r1_2d, r0_2d, scores_2d)


def _scores_body(u_ref, rel_ref, out_ref):
    out_ref[...] = lax.dot_general(u_ref[...], rel_ref[...],
                                   (((1,), (1,)), ((), ())),
                                   preferred_element_type=jnp.float32)


def _tc_scores(U, rel_emb):
    B = U.shape[0]
    return pl.pallas_call(
        _scores_body,
        out_shape=jax.ShapeDtypeStruct((B, rel_emb.shape[0]), jnp.float32),
    )(U, rel_emb)


def _dense_body(u_ref, ev0_ref, ev1_ref, ag1_ref, w0_ref, w_ref, b_ref,
                out_ref):
    bb = u_ref.shape[0]
    U = u_ref[...]
    W = w_ref[...]
    bias = b_ref[...]
    EV0 = ev0_ref[...]
    EV1 = ev1_ref[...]
    agg1 = ag1_ref[...]
    w0 = w0_ref[...]

    h1 = jax.nn.sigmoid(
        jnp.dot((EV1 + agg1).reshape(bb * 16, 128), W,
                preferred_element_type=jnp.float32) + bias
    ).reshape(bb, 16, 128)
    agg0 = jnp.sum(w0[..., None] * EV1, axis=1)
    h0 = jax.nn.sigmoid(
        jnp.dot(EV0 + agg0, W, preferred_element_type=jnp.float32) + bias)
    agg0b = jnp.sum(w0[..., None] * h1, axis=1)
    final = jnp.tanh(
        jnp.dot(h0 + agg0b, W, preferred_element_type=jnp.float32) + bias)
    out_ref[...] = jax.nn.sigmoid(jnp.sum(U * final, axis=1))[:, None]


def _tc_dense(U, EV0, EV1, AG1, w0, W, bvec):
    B = U.shape[0]
    bb = 128
    grid = B // bb
    return pl.pallas_call(
        _dense_body,
        grid=(grid,),
        in_specs=[
            pl.BlockSpec((bb, 128), lambda i: (i, 0)),
            pl.BlockSpec((bb, 128), lambda i: (i, 0)),
            pl.BlockSpec((bb, 16, 128), lambda i: (i, 0, 0)),
            pl.BlockSpec((bb, 16, 128), lambda i: (i, 0, 0)),
            pl.BlockSpec((bb, 16), lambda i: (i, 0)),
            pl.BlockSpec((128, 128), lambda i: (0, 0)),
            pl.BlockSpec((1, 128), lambda i: (0, 0)),
        ],
        out_specs=pl.BlockSpec((bb, 1), lambda i: (i, 0)),
        out_shape=jax.ShapeDtypeStruct((B, 1), jnp.float32),
    )(U, EV0, EV1, AG1, w0, W, bvec.reshape(1, 128))


def kernel(ent_emb, usr_emb, rel_emb, W, b, adj_ent, adj_rel, u, v):
    B = u.shape[0]
    n_nb = adj_ent.shape[1]

    e1, r0, U, EV0 = _multi_gather(
        [(adj_ent, v), (adj_rel, v), (usr_emb, u), (ent_emb, v)])
    e1f = e1.reshape(-1)
    e2, r1, EV1 = _multi_gather(
        [(adj_ent, e1f), (adj_rel, e1f), (ent_emb, e1f)])

    scores = _tc_scores(U, rel_emb)
    agg1, w0 = _sc_fused_agg(ent_emb, e2, r1, r0,
                             scores.reshape(-1, 128))

    out = _tc_dense(
        U, EV0,
        EV1.reshape(B, n_nb, 128),
        agg1.reshape(B, n_nb, 128),
        w0.reshape(B, n_nb),
        W, b)
    return out.reshape(B)


# speedup vs baseline: 1.6095x; 1.6095x over previous
import functools

import jax
import jax.numpy as jnp
from jax import lax
from jax.experimental import pallas as pl
from jax.experimental.pallas import tpu as pltpu
from jax.experimental.pallas import tpu_sc as plsc

_NC = 2
_NS = 16
_NW = _NC * _NS


def _multi_gather(pairs):
    specs = []
    ins = []
    scratch = []
    out_type = []
    for t, i in pairs:
        nrow = i.shape[0]
        d = t.shape[1]
        b_per_w = nrow // _NW
        cpw = min(128, b_per_w)
        nch = b_per_w // cpw
        wide = d % 128 == 0
        specs.append((d, b_per_w, cpw, nch, wide))
        scr = [
            pltpu.VMEM((nch, cpw), jnp.int32),
            pltpu.VMEM((cpw, d if wide else 128), t.dtype),
            pltpu.SemaphoreType.DMA,
        ]
        if wide:
            ins += [t, i.reshape(-1, cpw)]
            out_type.append(jax.ShapeDtypeStruct((nrow, d), t.dtype))
        else:
            assert d == 16
            ins += [t.reshape(-1, 128), i.reshape(-1, cpw)]
            out_type.append(
                jax.ShapeDtypeStruct((nrow * d // 128, 128), t.dtype))
            scr += [
                pltpu.VMEM((cpw,), jnp.int32),
                pltpu.VMEM((cpw * d // 128, 128), t.dtype),
            ]
        scratch.append(tuple(scr))

    n = len(specs)
    mesh = plsc.VectorSubcoreMesh(core_axis_name="c", subcore_axis_name="s")

    def body(*refs):
        tables = refs[0:2 * n:2]
        idxs = refs[1:2 * n:2]
        outs = refs[2 * n:3 * n]
        scr = refs[3 * n:]
        wid = lax.axis_index("s") * _NC + lax.axis_index("c")
        for g in range(n):
            d, b_per_w, cpw, nch, wide = specs[g]
            table, idx2d, out = tables[g], idxs[g], outs[g]
            idx_v = scr[g][0]
            pltpu.sync_copy(idx2d.at[pl.ds(wid * nch, nch)], idx_v)

            if wide:
                def chunk(c, _, table=table, out=out, idx_v=idx_v,
                          buf=scr[g][1], sem=scr[g][2],
                          base=wid * b_per_w, cpw=cpw):
                    pltpu.async_copy(table.at[idx_v.at[c]], buf, sem).wait()
                    pltpu.sync_copy(buf, out.at[pl.ds(base + c * cpw, cpw)])
                    return _
            else:
                def chunk(c, _, table=table, out=out, idx_v=idx_v,
                          buf=scr[g][1], sem=scr[g][2], idx8=scr[g][3],
                          obuf=scr[g][4], wrows=b_per_w * d // 128,
                          crows=cpw * d // 128, cpw=cpw):
                    lane = lax.iota(jnp.int32, 16)
                    for gg in range(cpw // 16):
                        tvec = idx_v[c, pl.ds(gg * 16, 16)]
                        idx8[pl.ds(gg * 16, 16)] = lax.shift_right_logical(
                            tvec, 3)
                    pltpu.async_copy(table.at[idx8], buf, sem).wait()
                    for gg in range(cpw // 16):
                        tvec = idx_v[c, pl.ds(gg * 16, 16)]
                        kbase = (tvec & 7) * 16
                        rows = lane + gg * 16
                        for j in range(16):
                            vals = plsc.load_gather(buf, [rows, kbase + j])
                            flat = rows * 16 + j
                            plsc.store_scatter(
                                obuf,
                                [lax.shift_right_logical(flat, 7), flat & 127],
                                vals)
                    pltpu.sync_copy(
                        obuf, out.at[pl.ds(wid * wrows + c * crows, crows)])
                    return _

            if nch == 1:
                chunk(0, None)
            else:
                lax.fori_loop(0, nch, chunk, None)

    f = pl.kernel(body, out_type=tuple(out_type), mesh=mesh,
                  scratch_types=tuple(scratch),
                  compiler_params=pltpu.CompilerParams(
                      needs_layout_passes=False))
    return f(*ins)


def _sc_fused_agg(ent_emb, e2_2d, r1_2d, r0_2d, scores_2d):
    mesh = plsc.VectorSubcoreMesh(core_axis_name="c", subcore_axis_name="s")
    CH = 128
    NCH = 64
    HALF = NCH // 2

    def body(ent, e2i, r1i, r0i, sco, agg_out, w0_out,
             idx_v, r1_v, r0_v, sco_v, stage, buf0, buf1, w0_v,
             sem0, sem1):
        wid = lax.axis_index("s") * _NC + lax.axis_index("c")
        lane = lax.iota(jnp.int32, 16)
        pltpu.sync_copy(e2i.at[pl.ds(wid * NCH, NCH)], idx_v)
        pltpu.sync_copy(r1i.at[pl.ds(wid * NCH, NCH)], r1_v)
        pltpu.sync_copy(r0i.at[pl.ds(wid * 4, 4)], r0_v)
        pltpu.sync_copy(sco.at[pl.ds(wid * 8, 8)], sco_v)

        def softmax16(svals):
            m = jnp.max(svals)
            es = jnp.exp(svals - m)
            return es / jnp.sum(es)

        def w0_item(i, _):
            rv = r0_v[i >> 3, pl.ds(pl.multiple_of((i & 7) * 16, 16), 16)]
            srow = jnp.broadcast_to(i >> 2, (16,))
            svals = plsc.load_gather(sco_v, [srow, (i & 3) * 32 + rv])
            w0_v[i >> 3, pl.ds(pl.multiple_of((i & 7) * 16, 16), 16)] = (
                softmax16(svals))
            return _
        lax.fori_loop(0, 32, w0_item, None)
        pltpu.sync_copy(w0_v, w0_out.at[pl.ds(wid * 4, 4)])

        bufs = (buf0, buf1)
        sems = (sem0, sem1)

        def issue(c, sub):
            pltpu.async_copy(ent.at[idx_v.at[jnp.minimum(c, NCH - 1)]],
                             bufs[sub], sems[sub])

        def process(c, cl, sub):
            buf = bufs[sub]
            item = c >> 1

            def group(g, _):
                coff = pl.multiple_of(g * 16, 16)
                rv = r1_v[c, pl.ds(coff, 16)]
                srow = jnp.broadcast_to(item >> 2, (16,))
                svals = plsc.load_gather(sco_v, [srow, (item & 3) * 32 + rv])
                w = softmax16(svals)
                srow16 = (cl >> 1) * 16 + (c & 1) * 8 + g
                wks = [jnp.broadcast_to(w[k], (16,)) for k in range(16)]
                for j in range(8):
                    parts = [
                        wks[k] * buf[g * 16 + k, pl.ds(j * 16, 16)]
                        for k in range(16)]
                    for step in (8, 4, 2, 1):
                        parts = [parts[t] + parts[t + step]
                                 for t in range(step)]
                    stage[srow16, pl.ds(j * 16, 16)] = parts[0]
                return _
            lax.fori_loop(0, 8, group, None)

        issue(0, 0)
        issue(1, 1)
        for h in range(2):
            def pair(p, _, h=h):
                cl = 2 * p
                c = h * HALF + cl
                for sub in range(2):
                    pltpu.make_async_copy(ent.at[idx_v.at[0]], bufs[sub],
                                          sems[sub]).wait()
                    process(c + sub, cl + sub, sub)
                    issue(c + sub + 2, sub)
                return _
            lax.fori_loop(0, HALF // 2, pair, None)
            pltpu.sync_copy(
                stage, agg_out.at[pl.ds(wid * 512 + h * 256, 256)])
        pltpu.make_async_copy(ent.at[idx_v.at[0]], bufs[0], sems[0]).wait()
        pltpu.make_async_copy(ent.at[idx_v.at[0]], bufs[1], sems[1]).wait()

    f = pl.kernel(
        body,
        out_type=(jax.ShapeDtypeStruct((16384, 128), jnp.float32),
                  jax.ShapeDtypeStruct((128, 128), jnp.float32)),
        mesh=mesh,
        scratch_types=(
            pltpu.VMEM((NCH, CH), jnp.int32),
            pltpu.VMEM((NCH, CH), jnp.int32),
            pltpu.VMEM((4, 128), jnp.int32),
            pltpu.VMEM((8, 128), jnp.float32),
            pltpu.VMEM((256, 128), jnp.float32),
            pltpu.VMEM((CH, 128), jnp.float32),
            pltpu.VMEM((CH, 128), jnp.float32),
            pltpu.VMEM((4, 128), jnp.float32),
            pltpu.SemaphoreType.DMA,
            pltpu.SemaphoreType.DMA,
        ),
        compiler_params=pltpu.CompilerParams(needs_layout_passes=False))
    return f(ent_emb, e2_2d, r1_2d, r0_2d, scores_2d)


def _scores_body(u_ref, rel_ref, out_ref):
    out_ref[...] = lax.dot_general(u_ref[...], rel_ref[...],
                                   (((1,), (1,)), ((), ())),
                                   preferred_element_type=jnp.float32)


def _tc_scores(U, rel_emb):
    B = U.shape[0]
    return pl.pallas_call(
        _scores_body,
        out_shape=jax.ShapeDtypeStruct((B, rel_emb.shape[0]), jnp.float32),
    )(U, rel_emb)


def _dense_body(u_ref, ev0_ref, ev1_ref, ag1_ref, w0_ref, w_ref, b_ref,
                out_ref):
    bb = u_ref.shape[0]
    U = u_ref[...]
    W = w_ref[...]
    bias = b_ref[...]
    EV0 = ev0_ref[...]
    EV1 = ev1_ref[...]
    agg1 = ag1_ref[...]
    w0 = w0_ref[...]

    h1 = jax.nn.sigmoid(
        jnp.dot((EV1 + agg1).reshape(bb * 16, 128), W,
                preferred_element_type=jnp.float32) + bias
    ).reshape(bb, 16, 128)
    agg0 = jnp.sum(w0[..., None] * EV1, axis=1)
    h0 = jax.nn.sigmoid(
        jnp.dot(EV0 + agg0, W, preferred_element_type=jnp.float32) + bias)
    agg0b = jnp.sum(w0[..., None] * h1, axis=1)
    final = jnp.tanh(
        jnp.dot(h0 + agg0b, W, preferred_element_type=jnp.float32) + bias)
    out_ref[...] = jax.nn.sigmoid(jnp.sum(U * final, axis=1))[:, None]


def _tc_dense(U, EV0, EV1, AG1, w0, W, bvec):
    B = U.shape[0]
    bb = 128
    grid = B // bb
    return pl.pallas_call(
        _dense_body,
        grid=(grid,),
        in_specs=[
            pl.BlockSpec((bb, 128), lambda i: (i, 0)),
            pl.BlockSpec((bb, 128), lambda i: (i, 0)),
            pl.BlockSpec((bb, 16, 128), lambda i: (i, 0, 0)),
            pl.BlockSpec((bb, 16, 128), lambda i: (i, 0, 0)),
            pl.BlockSpec((bb, 16), lambda i: (i, 0)),
            pl.BlockSpec((128, 128), lambda i: (0, 0)),
            pl.BlockSpec((1, 128), lambda i: (0, 0)),
        ],
        out_specs=pl.BlockSpec((bb, 1), lambda i: (i, 0)),
        out_shape=jax.ShapeDtypeStruct((B, 1), jnp.float32),
    )(U, EV0, EV1, AG1, w0, W, bvec.reshape(1, 128))


def kernel(ent_emb, usr_emb, rel_emb, W, b, adj_ent, adj_rel, u, v):
    B = u.shape[0]
    n_nb = adj_ent.shape[1]

    e1, r0, U, EV0 = _multi_gather(
        [(adj_ent, v), (adj_rel, v), (usr_emb, u), (ent_emb, v)])
    e1f = e1.reshape(-1)
    e2, r1, EV1 = _multi_gather(
        [(adj_ent, e1f), (adj_rel, e1f), (ent_emb, e1f)])

    scores = _tc_scores(U, rel_emb)
    agg1, w0 = _sc_fused_agg(ent_emb, e2, r1, r0,
                             scores.reshape(-1, 128))

    out = _tc_dense(
        U, EV0,
        EV1.reshape(B, n_nb, 128),
        agg1.reshape(B, n_nb, 128),
        w0.reshape(B, n_nb),
        W, b)
    return out.reshape(B)


# speedup vs baseline: 1.6158x; 1.0039x over previous
import functools

import jax
import jax.numpy as jnp
from jax import lax
from jax.experimental import pallas as pl
from jax.experimental.pallas import tpu as pltpu
from jax.experimental.pallas import tpu_sc as plsc

_NC = 2
_NS = 16
_NW = _NC * _NS


def _multi_gather(pairs):
    specs = []
    ins = []
    scratch = []
    out_type = []
    for t, i in pairs:
        nrow = i.shape[0]
        d = t.shape[1]
        b_per_w = nrow // _NW
        cpw = min(128, b_per_w)
        nch = b_per_w // cpw
        wide = d % 128 == 0
        specs.append((d, b_per_w, cpw, nch, wide))
        scr = [
            pltpu.VMEM((nch, cpw), jnp.int32),
            pltpu.VMEM((cpw, d if wide else 128), t.dtype),
            pltpu.SemaphoreType.DMA,
        ]
        if wide:
            ins += [t, i.reshape(-1, cpw)]
            out_type.append(jax.ShapeDtypeStruct((nrow, d), t.dtype))
        else:
            assert d == 16
            ins += [t.reshape(-1, 128), i.reshape(-1, cpw)]
            out_type.append(
                jax.ShapeDtypeStruct((nrow * d // 128, 128), t.dtype))
            scr += [
                pltpu.VMEM((cpw,), jnp.int32),
                pltpu.VMEM((cpw * d // 128, 128), t.dtype),
            ]
        scratch.append(tuple(scr))

    n = len(specs)
    mesh = plsc.VectorSubcoreMesh(core_axis_name="c", subcore_axis_name="s")

    def body(*refs):
        tables = refs[0:2 * n:2]
        idxs = refs[1:2 * n:2]
        outs = refs[2 * n:3 * n]
        scr = refs[3 * n:]
        wid = lax.axis_index("s") * _NC + lax.axis_index("c")
        for g in range(n):
            d, b_per_w, cpw, nch, wide = specs[g]
            table, idx2d, out = tables[g], idxs[g], outs[g]
            idx_v = scr[g][0]
            pltpu.sync_copy(idx2d.at[pl.ds(wid * nch, nch)], idx_v)

            if wide:
                def chunk(c, _, table=table, out=out, idx_v=idx_v,
                          buf=scr[g][1], sem=scr[g][2],
                          base=wid * b_per_w, cpw=cpw):
                    pltpu.async_copy(table.at[idx_v.at[c]], buf, sem).wait()
                    pltpu.sync_copy(buf, out.at[pl.ds(base + c * cpw, cpw)])
                    return _
            else:
                def chunk(c, _, table=table, out=out, idx_v=idx_v,
                          buf=scr[g][1], sem=scr[g][2], idx8=scr[g][3],
                          obuf=scr[g][4], wrows=b_per_w * d // 128,
                          crows=cpw * d // 128, cpw=cpw):
                    lane = lax.iota(jnp.int32, 16)
                    for gg in range(cpw // 16):
                        tvec = idx_v[c, pl.ds(gg * 16, 16)]
                        idx8[pl.ds(gg * 16, 16)] = lax.shift_right_logical(
                            tvec, 3)
                    pltpu.async_copy(table.at[idx8], buf, sem).wait()
                    for gg in range(cpw // 16):
                        tvec = idx_v[c, pl.ds(gg * 16, 16)]
                        kbase = (tvec & 7) * 16
                        rows = lane + gg * 16
                        for j in range(16):
                            vals = plsc.load_gather(buf, [rows, kbase + j])
                            flat = rows * 16 + j
                            plsc.store_scatter(
                                obuf,
                                [lax.shift_right_logical(flat, 7), flat & 127],
                                vals)
                    pltpu.sync_copy(
                        obuf, out.at[pl.ds(wid * wrows + c * crows, crows)])
                    return _

            if nch == 1:
                chunk(0, None)
            else:
                lax.fori_loop(0, nch, chunk, None)

    f = pl.kernel(body, out_type=tuple(out_type), mesh=mesh,
                  scratch_types=tuple(scratch),
                  compiler_params=pltpu.CompilerParams(
                      needs_layout_passes=False))
    return f(*ins)


def _sc_fused_agg(ent_emb, e2_2d, r1_2d, r0_2d, scores_2d):
    mesh = plsc.VectorSubcoreMesh(core_axis_name="c", subcore_axis_name="s")
    CH = 128
    NCH = 64
    HALF = NCH // 2

    def body(ent, e2i, r1i, r0i, sco, agg_out, w0_out,
             idx_v, r1_v, r0_v, sco_v, stage, buf0, buf1, w0_v,
             sem0, sem1):
        wid = lax.axis_index("s") * _NC + lax.axis_index("c")
        lane = lax.iota(jnp.int32, 16)
        pltpu.sync_copy(e2i.at[pl.ds(wid * NCH, NCH)], idx_v)
        pltpu.sync_copy(r1i.at[pl.ds(wid * NCH, NCH)], r1_v)
        pltpu.sync_copy(r0i.at[pl.ds(wid * 4, 4)], r0_v)
        pltpu.sync_copy(sco.at[pl.ds(wid * 8, 8)], sco_v)

        def softmax16(svals):
            m = jnp.max(svals)
            es = jnp.exp(svals - m)
            return es / jnp.sum(es)

        @plsc.parallel_loop(0, 32, unroll=4)
        def w0_item(i):
            rv = r0_v[i >> 3, pl.ds(pl.multiple_of((i & 7) * 16, 16), 16)]
            srow = jnp.broadcast_to(i >> 2, (16,))
            svals = plsc.load_gather(sco_v, [srow, (i & 3) * 32 + rv])
            w0_v[i >> 3, pl.ds(pl.multiple_of((i & 7) * 16, 16), 16)] = (
                softmax16(svals))
        pltpu.sync_copy(w0_v, w0_out.at[pl.ds(wid * 4, 4)])

        bufs = (buf0, buf1)
        sems = (sem0, sem1)

        def issue(c, sub):
            pltpu.async_copy(ent.at[idx_v.at[jnp.minimum(c, NCH - 1)]],
                             bufs[sub], sems[sub])

        def process(c, cl, sub):
            buf = bufs[sub]
            item = c >> 1

            @plsc.parallel_loop(0, 8, unroll=2)
            def group(g):
                coff = pl.multiple_of(g * 16, 16)
                rv = r1_v[c, pl.ds(coff, 16)]
                srow = jnp.broadcast_to(item >> 2, (16,))
                svals = plsc.load_gather(sco_v, [srow, (item & 3) * 32 + rv])
                w = softmax16(svals)
                srow16 = (cl >> 1) * 16 + (c & 1) * 8 + g
                wks = [jnp.broadcast_to(w[k], (16,)) for k in range(16)]
                for j in range(8):
                    parts = [
                        wks[k] * buf[g * 16 + k, pl.ds(j * 16, 16)]
                        for k in range(16)]
                    for step in (8, 4, 2, 1):
                        parts = [parts[t] + parts[t + step]
                                 for t in range(step)]
                    stage[srow16, pl.ds(j * 16, 16)] = parts[0]

        issue(0, 0)
        issue(1, 1)
        for h in range(2):
            def pair(p, _, h=h):
                cl = 2 * p
                c = h * HALF + cl
                for sub in range(2):
                    pltpu.make_async_copy(ent.at[idx_v.at[0]], bufs[sub],
                                          sems[sub]).wait()
                    process(c + sub, cl + sub, sub)
                    issue(c + sub + 2, sub)
                return _
            lax.fori_loop(0, HALF // 2, pair, None)
            pltpu.sync_copy(
                stage, agg_out.at[pl.ds(wid * 512 + h * 256, 256)])
        pltpu.make_async_copy(ent.at[idx_v.at[0]], bufs[0], sems[0]).wait()
        pltpu.make_async_copy(ent.at[idx_v.at[0]], bufs[1], sems[1]).wait()

    f = pl.kernel(
        body,
        out_type=(jax.ShapeDtypeStruct((16384, 128), jnp.float32),
                  jax.ShapeDtypeStruct((128, 128), jnp.float32)),
        mesh=mesh,
        scratch_types=(
            pltpu.VMEM((NCH, CH), jnp.int32),
            pltpu.VMEM((NCH, CH), jnp.int32),
            pltpu.VMEM((4, 128), jnp.int32),
            pltpu.VMEM((8, 128), jnp.float32),
            pltpu.VMEM((256, 128), jnp.float32),
            pltpu.VMEM((CH, 128), jnp.float32),
            pltpu.VMEM((CH, 128), jnp.float32),
            pltpu.VMEM((4, 128), jnp.float32),
            pltpu.SemaphoreType.DMA,
            pltpu.SemaphoreType.DMA,
        ),
        compiler_params=pltpu.CompilerParams(needs_layout_passes=False))
    return f(ent_emb, e2_2d, r1_2d, r0_2d, scores_2d)


def _scores_body(u_ref, rel_ref, out_ref):
    out_ref[...] = lax.dot_general(u_ref[...], rel_ref[...],
                                   (((1,), (1,)), ((), ())),
                                   preferred_element_type=jnp.float32)


def _tc_scores(U, rel_emb):
    B = U.shape[0]
    return pl.pallas_call(
        _scores_body,
        out_shape=jax.ShapeDtypeStruct((B, rel_emb.shape[0]), jnp.float32),
    )(U, rel_emb)


def _dense_body(u_ref, ev0_ref, ev1_ref, ag1_ref, w0_ref, w_ref, b_ref,
                out_ref):
    bb = u_ref.shape[0]
    U = u_ref[...]
    W = w_ref[...]
    bias = b_ref[...]
    EV0 = ev0_ref[...]
    EV1 = ev1_ref[...]
    agg1 = ag1_ref[...]
    w0 = w0_ref[...]

    h1 = jax.nn.sigmoid(
        jnp.dot((EV1 + agg1).reshape(bb * 16, 128), W,
                preferred_element_type=jnp.float32) + bias
    ).reshape(bb, 16, 128)
    agg0 = jnp.sum(w0[..., None] * EV1, axis=1)
    h0 = jax.nn.sigmoid(
        jnp.dot(EV0 + agg0, W, preferred_element_type=jnp.float32) + bias)
    agg0b = jnp.sum(w0[..., None] * h1, axis=1)
    final = jnp.tanh(
        jnp.dot(h0 + agg0b, W, preferred_element_type=jnp.float32) + bias)
    out_ref[...] = jax.nn.sigmoid(jnp.sum(U * final, axis=1))[:, None]


def _tc_dense(U, EV0, EV1, AG1, w0, W, bvec):
    B = U.shape[0]
    bb = 128
    grid = B // bb
    return pl.pallas_call(
        _dense_body,
        grid=(grid,),
        in_specs=[
            pl.BlockSpec((bb, 128), lambda i: (i, 0)),
            pl.BlockSpec((bb, 128), lambda i: (i, 0)),
            pl.BlockSpec((bb, 16, 128), lambda i: (i, 0, 0)),
            pl.BlockSpec((bb, 16, 128), lambda i: (i, 0, 0)),
            pl.BlockSpec((bb, 16), lambda i: (i, 0)),
            pl.BlockSpec((128, 128), lambda i: (0, 0)),
            pl.BlockSpec((1, 128), lambda i: (0, 0)),
        ],
        out_specs=pl.BlockSpec((bb, 1), lambda i: (i, 0)),
        out_shape=jax.ShapeDtypeStruct((B, 1), jnp.float32),
    )(U, EV0, EV1, AG1, w0, W, bvec.reshape(1, 128))


def kernel(ent_emb, usr_emb, rel_emb, W, b, adj_ent, adj_rel, u, v):
    B = u.shape[0]
    n_nb = adj_ent.shape[1]

    e1, r0, U, EV0 = _multi_gather(
        [(adj_ent, v), (adj_rel, v), (usr_emb, u), (ent_emb, v)])
    e1f = e1.reshape(-1)
    e2, r1, EV1 = _multi_gather(
        [(adj_ent, e1f), (adj_rel, e1f), (ent_emb, e1f)])

    scores = _tc_scores(U, rel_emb)
    agg1, w0 = _sc_fused_agg(ent_emb, e2, r1, r0,
                             scores.reshape(-1, 128))

    out = _tc_dense(
        U, EV0,
        EV1.reshape(B, n_nb, 128),
        agg1.reshape(B, n_nb, 128),
        w0.reshape(B, n_nb),
        W, b)
    return out.reshape(B)


# speedup vs baseline: 1.7832x; 1.1036x over previous
import functools

import jax
import jax.numpy as jnp
from jax import lax
from jax.experimental import pallas as pl
from jax.experimental.pallas import tpu as pltpu
from jax.experimental.pallas import tpu_sc as plsc

_NC = 2
_NS = 16
_NW = _NC * _NS


def _multi_gather(pairs):
    specs = []
    ins = []
    scratch = []
    out_type = []
    for t, i in pairs:
        nrow = i.shape[0]
        d = t.shape[1]
        b_per_w = nrow // _NW
        cpw = min(128, b_per_w)
        nch = b_per_w // cpw
        wide = d % 128 == 0
        specs.append((d, b_per_w, cpw, nch, wide))
        scr = [
            pltpu.VMEM((nch, cpw), jnp.int32),
            pltpu.VMEM((cpw, d if wide else 128), t.dtype),
            pltpu.SemaphoreType.DMA,
        ]
        if wide:
            ins += [t, i.reshape(-1, cpw)]
            out_type.append(jax.ShapeDtypeStruct((nrow, d), t.dtype))
        else:
            assert d == 16
            ins += [t.reshape(-1, 128), i.reshape(-1, cpw)]
            out_type.append(
                jax.ShapeDtypeStruct((nrow * d // 128, 128), t.dtype))
            scr += [
                pltpu.VMEM((cpw,), jnp.int32),
                pltpu.VMEM((cpw * d // 128, 128), t.dtype),
            ]
        scratch.append(tuple(scr))

    n = len(specs)
    mesh = plsc.VectorSubcoreMesh(core_axis_name="c", subcore_axis_name="s")

    def body(*refs):
        tables = refs[0:2 * n:2]
        idxs = refs[1:2 * n:2]
        outs = refs[2 * n:3 * n]
        scr = refs[3 * n:]
        wid = lax.axis_index("s") * _NC + lax.axis_index("c")
        for g in range(n):
            d, b_per_w, cpw, nch, wide = specs[g]
            table, idx2d, out = tables[g], idxs[g], outs[g]
            idx_v = scr[g][0]
            pltpu.sync_copy(idx2d.at[pl.ds(wid * nch, nch)], idx_v)

            if wide:
                def chunk(c, _, table=table, out=out, idx_v=idx_v,
                          buf=scr[g][1], sem=scr[g][2],
                          base=wid * b_per_w, cpw=cpw):
                    pltpu.async_copy(table.at[idx_v.at[c]], buf, sem).wait()
                    pltpu.sync_copy(buf, out.at[pl.ds(base + c * cpw, cpw)])
                    return _
            else:
                def chunk(c, _, table=table, out=out, idx_v=idx_v,
                          buf=scr[g][1], sem=scr[g][2], idx8=scr[g][3],
                          obuf=scr[g][4], wrows=b_per_w * d // 128,
                          crows=cpw * d // 128, cpw=cpw):
                    lane = lax.iota(jnp.int32, 16)
                    for gg in range(cpw // 16):
                        tvec = idx_v[c, pl.ds(gg * 16, 16)]
                        idx8[pl.ds(gg * 16, 16)] = lax.shift_right_logical(
                            tvec, 3)
                    pltpu.async_copy(table.at[idx8], buf, sem).wait()
                    for gg in range(cpw // 16):
                        tvec = idx_v[c, pl.ds(gg * 16, 16)]
                        kbase = (tvec & 7) * 16
                        rows = lane + gg * 16
                        for j in range(16):
                            vals = plsc.load_gather(buf, [rows, kbase + j])
                            flat = rows * 16 + j
                            plsc.store_scatter(
                                obuf,
                                [lax.shift_right_logical(flat, 7), flat & 127],
                                vals)
                    pltpu.sync_copy(
                        obuf, out.at[pl.ds(wid * wrows + c * crows, crows)])
                    return _

            if nch == 1:
                chunk(0, None)
            else:
                lax.fori_loop(0, nch, chunk, None)

    f = pl.kernel(body, out_type=tuple(out_type), mesh=mesh,
                  scratch_types=tuple(scratch),
                  compiler_params=pltpu.CompilerParams(
                      needs_layout_passes=False))
    return f(*ins)


def _sc_fused_agg(ent_emb, e2_2d, r1_2d, r0_2d, scores_2d):
    mesh = plsc.VectorSubcoreMesh(core_axis_name="c", subcore_axis_name="s")
    CH = 128
    NCH = 64
    HALF = NCH // 2

    def body(ent, e2i, r1i, r0i, sco, agg_out, w0_out,
             idx_v, r1_v, r0_v, sco_v, stage, buf0, buf1, w0_v,
             sem0, sem1):
        wid = lax.axis_index("s") * _NC + lax.axis_index("c")
        lane = lax.iota(jnp.int32, 16)
        pltpu.sync_copy(e2i.at[pl.ds(wid * NCH, NCH)], idx_v)
        pltpu.sync_copy(r1i.at[pl.ds(wid * NCH, NCH)], r1_v)
        pltpu.sync_copy(r0i.at[pl.ds(wid * 4, 4)], r0_v)
        pltpu.sync_copy(sco.at[pl.ds(wid * 8, 8)], sco_v)

        def softmax16(svals):
            m = jnp.max(svals)
            es = jnp.exp(svals - m)
            return es / jnp.sum(es)

        @plsc.parallel_loop(0, 32, unroll=4)
        def w0_item(i):
            rv = r0_v[i >> 3, pl.ds(pl.multiple_of((i & 7) * 16, 16), 16)]
            srow = jnp.broadcast_to(i >> 2, (16,))
            svals = plsc.load_gather(sco_v, [srow, (i & 3) * 32 + rv])
            w0_v[i >> 3, pl.ds(pl.multiple_of((i & 7) * 16, 16), 16)] = (
                softmax16(svals))
        pltpu.sync_copy(w0_v, w0_out.at[pl.ds(wid * 4, 4)])

        bufs = (buf0, buf1)
        sems = (sem0, sem1)

        def issue(c, sub):
            pltpu.async_copy(ent.at[idx_v.at[jnp.minimum(c, NCH - 1)]],
                             bufs[sub], sems[sub])

        def process(c, cl, sub):
            buf = bufs[sub]
            item = c >> 1

            @plsc.parallel_loop(0, 8, unroll=2)
            def group(g):
                coff = pl.multiple_of(g * 16, 16)
                rv = r1_v[c, pl.ds(coff, 16)]
                srow = jnp.broadcast_to(item >> 2, (16,))
                svals = plsc.load_gather(sco_v, [srow, (item & 3) * 32 + rv])
                w = softmax16(svals)
                srow16 = (cl >> 1) * 16 + (c & 1) * 8 + g
                wks = [jnp.broadcast_to(w[k], (16,)) for k in range(16)]
                for j in range(8):
                    parts = [
                        wks[k] * buf[g * 16 + k, pl.ds(j * 16, 16)]
                        for k in range(16)]
                    for step in (8, 4, 2, 1):
                        parts = [parts[t] + parts[t + step]
                                 for t in range(step)]
                    stage[srow16, pl.ds(j * 16, 16)] = parts[0]

        issue(0, 0)
        issue(1, 1)
        for h in range(2):
            def pair(p, _, h=h):
                cl = 2 * p
                c = h * HALF + cl
                for sub in range(2):
                    pltpu.make_async_copy(ent.at[idx_v.at[0]], bufs[sub],
                                          sems[sub]).wait()
                    process(c + sub, cl + sub, sub)
                    issue(c + sub + 2, sub)
                return _
            lax.fori_loop(0, HALF // 2, pair, None)
            pltpu.sync_copy(
                stage, agg_out.at[pl.ds(wid * 512 + h * 256, 256)])
        pltpu.make_async_copy(ent.at[idx_v.at[0]], bufs[0], sems[0]).wait()
        pltpu.make_async_copy(ent.at[idx_v.at[0]], bufs[1], sems[1]).wait()

    f = pl.kernel(
        body,
        out_type=(jax.ShapeDtypeStruct((16384, 128), jnp.float32),
                  jax.ShapeDtypeStruct((128, 128), jnp.float32)),
        mesh=mesh,
        scratch_types=(
            pltpu.VMEM((NCH, CH), jnp.int32),
            pltpu.VMEM((NCH, CH), jnp.int32),
            pltpu.VMEM((4, 128), jnp.int32),
            pltpu.VMEM((8, 128), jnp.float32),
            pltpu.VMEM((256, 128), jnp.float32),
            pltpu.VMEM((CH, 128), jnp.float32),
            pltpu.VMEM((CH, 128), jnp.float32),
            pltpu.VMEM((4, 128), jnp.float32),
            pltpu.SemaphoreType.DMA,
            pltpu.SemaphoreType.DMA,
        ),
        compiler_params=pltpu.CompilerParams(needs_layout_passes=False))
    return f(ent_emb, e2_2d, r1_2d, r0_2d, scores_2d)


def _sc_merged(ent_emb, usr_emb, rel_emb, adjE8, adjR8, u32, v32):
    mesh = plsc.VectorSubcoreMesh(core_axis_name="c", subcore_axis_name="s")

    def body(ent, usr, rel, adjE, adjR, u_in, v_in,
             u_out, ev0_out, ev1_out, agg_out, w0_out,
             v_v, u_v, sidx, ubuf, e0buf,
             abufA0, abufA1, abufB0, abufB1,
             e1_v, r0_v, rel_v, sco_v, idx64a, idx64b,
             e2_v, r1_v, buf0, buf1, stage, w0_v,
             semA0, semA1, semB0, semB1, sem0, sem1):
        wid = lax.axis_index("s") * _NC + lax.axis_index("c")
        lane = lax.iota(jnp.int32, 16)
        pltpu.sync_copy(v_in.at[pl.ds(wid, 1)], v_v)
        pltpu.sync_copy(u_in.at[pl.ds(wid, 1)], u_v)
        pltpu.sync_copy(rel, rel_v)

        for g in range(2):
            t = v_v[0, pl.ds(g * 16, 16)]
            sidx[pl.ds(g * 16, 16)] = lax.shift_right_logical(t, 3)
        pltpu.async_copy(adjE.at[sidx], abufA0.at[pl.ds(0, 32)], semA0)
        pltpu.async_copy(adjR.at[sidx], abufB0.at[pl.ds(0, 32)], semB0)
        pltpu.async_copy(usr.at[u_v.at[0]], ubuf, sem0)
        pltpu.async_copy(ent.at[v_v.at[0]], e0buf, sem1)
        pltpu.make_async_copy(adjE.at[sidx], abufA0.at[pl.ds(0, 32)],
                              semA0).wait()
        pltpu.make_async_copy(adjR.at[sidx], abufB0.at[pl.ds(0, 32)],
                              semB0).wait()
        pltpu.make_async_copy(usr.at[u_v.at[0]], ubuf, sem0).wait()
        pltpu.make_async_copy(ent.at[v_v.at[0]], e0buf, sem1).wait()
        pltpu.sync_copy(ubuf, u_out.at[pl.ds(wid * 32, 32)])
        pltpu.sync_copy(e0buf, ev0_out.at[pl.ds(wid * 32, 32)])
        for gg in range(2):
            tv = v_v[0, pl.ds(gg * 16, 16)]
            kbase = (tv & 7) * 16
            rows = lane + gg * 16
            for j in range(16):
                flat = rows * 16 + j
                plsc.store_scatter(
                    e1_v, [lax.shift_right_logical(flat, 7), flat & 127],
                    plsc.load_gather(abufA0, [rows, kbase + j]))
                plsc.store_scatter(
                    r0_v, [lax.shift_right_logical(flat, 7), flat & 127],
                    plsc.load_gather(abufB0, [rows, kbase + j]))

        idxs = (idx64a, idx64b)
        abufsA = (abufA0, abufA1)
        abufsB = (abufB0, abufB1)
        semsA = (semA0, semA1)
        semsB = (semB0, semB1)
        bufs = (buf0, buf1)
        sems = (sem0, sem1)

        def shift_chunk(c, dst):
            for g in range(2):
                off = pl.multiple_of((c & 3) * 32 + g * 16, 16)
                t = e1_v[lax.shift_right_logical(c, 2), pl.ds(off, 16)]
                dst[pl.ds(g * 16, 16)] = lax.shift_right_logical(t, 3)

        def issueC(c, sub):
            pltpu.async_copy(adjE.at[idxs[sub]], abufsA[sub], semsA[sub])
            pltpu.async_copy(adjR.at[idxs[sub]], abufsB[sub], semsB[sub])

        def ev1_issue(c, sub):
            pltpu.async_copy(ent.at[e1_v.at[c]], bufs[sub], sems[sub])

        shift_chunk(0, idx64a)
        issueC(0, 0)
        shift_chunk(1, idx64b)
        issueC(1, 1)
        ev1_issue(0, 0)
        ev1_issue(1, 1)

        @plsc.parallel_loop(0, 32, unroll=2)
        def score_item(i):
            urow = [ubuf[i, pl.ds(jj * 16, 16)] for jj in range(8)]
            svec = [jnp.zeros((16,), jnp.float32) for _ in range(2)]
            for r in range(32):
                ps = [urow[jj] * rel_v[r, pl.ds(jj * 16, 16)]
                      for jj in range(8)]
                for step in (4, 2, 1):
                    ps = [ps[t] + ps[t + step] for t in range(step)]
                s = jnp.sum(ps[0])
                svec[r >> 4] = jnp.where(lane == (r & 15), s, svec[r >> 4])
            base = pl.multiple_of((i & 3) * 32, 32)
            sco_v[i >> 2, pl.ds(base, 16)] = svec[0]
            sco_v[i >> 2, pl.ds(pl.multiple_of(base + 16, 16), 16)] = svec[1]

        def cpair(p, _):
            for sub in range(2):
                c = 2 * p + sub
                pltpu.make_async_copy(adjE.at[idxs[sub]], abufsA[sub],
                                      semsA[sub]).wait()
                pltpu.make_async_copy(adjR.at[idxs[sub]], abufsB[sub],
                                      semsB[sub]).wait()
                crow = lax.shift_right_logical(c, 2)
                cofs = (c & 3) * 32
                for gg in range(2):
                    off = pl.multiple_of(cofs + gg * 16, 16)
                    tv = e1_v[crow, pl.ds(off, 16)]
                    kbase = (tv & 7) * 16
                    rows = lane + gg * 16
                    for j in range(16):
                        flat = rows * 16 + j
                        er = c * 4 + lax.shift_right_logical(flat, 7)
                        plsc.store_scatter(
                            e2_v, [er, flat & 127],
                            plsc.load_gather(abufsA[sub],
                                             [rows, kbase + j]))
                        plsc.store_scatter(
                            r1_v, [er, flat & 127],
                            plsc.load_gather(abufsB[sub],
                                             [rows, kbase + j]))
                nxt = jnp.minimum(c + 2, 15)
                shift_chunk(nxt, idxs[sub])
                issueC(nxt, sub)
            return _
        lax.fori_loop(0, 8, cpair, None)
        for sub in range(2):
            pltpu.make_async_copy(adjE.at[idxs[sub]], abufsA[sub],
                                  semsA[sub]).wait()
            pltpu.make_async_copy(adjR.at[idxs[sub]], abufsB[sub],
                                  semsB[sub]).wait()

        def ev1_wait(sub):
            pltpu.make_async_copy(ent.at[e1_v.at[0]], bufs[sub],
                                  sems[sub]).wait()

        for c in range(4):
            sub = c & 1
            ev1_wait(sub)
            pltpu.sync_copy(bufs[sub],
                            ev1_out.at[pl.ds(wid * 512 + c * 128, 128)])
            if c + 2 < 4:
                ev1_issue(c + 2, sub)

        def softmax16(svals):
            m = jnp.max(svals)
            es = jnp.exp(svals - m)
            return es / jnp.sum(es)

        @plsc.parallel_loop(0, 32, unroll=4)
        def w0_item(i):
            off = pl.multiple_of((i & 7) * 16, 16)
            rv = r0_v[i >> 3, pl.ds(off, 16)]
            srow = jnp.broadcast_to(i >> 2, (16,))
            svals = plsc.load_gather(sco_v, [srow, (i & 3) * 32 + rv])
            w0_v[i >> 3, pl.ds(off, 16)] = softmax16(svals)
        pltpu.sync_copy(w0_v, w0_out.at[pl.ds(wid * 4, 4)])

        def issueD(c, sub):
            pltpu.async_copy(ent.at[e2_v.at[jnp.minimum(c, 63)]],
                             bufs[sub], sems[sub])

        def processD(c, cl, sub):
            buf = bufs[sub]
            item = c >> 1

            @plsc.parallel_loop(0, 8, unroll=2)
            def group(g):
                coff = pl.multiple_of(g * 16, 16)
                rv = r1_v[c, pl.ds(coff, 16)]
                srow = jnp.broadcast_to(item >> 2, (16,))
                svals = plsc.load_gather(sco_v, [srow, (item & 3) * 32 + rv])
                w = softmax16(svals)
                srow16 = (cl >> 1) * 16 + (c & 1) * 8 + g
                wks = [jnp.broadcast_to(w[k], (16,)) for k in range(16)]
                for j in range(8):
                    parts = [
                        wks[k] * buf[g * 16 + k, pl.ds(j * 16, 16)]
                        for k in range(16)]
                    for step in (8, 4, 2, 1):
                        parts = [parts[t] + parts[t + step]
                                 for t in range(step)]
                    stage[srow16, pl.ds(j * 16, 16)] = parts[0]

        issueD(0, 0)
        issueD(1, 1)
        for h in range(2):
            def dpair(p, _, h=h):
                cl = 2 * p
                c = h * 32 + cl
                for sub in range(2):
                    pltpu.make_async_copy(ent.at[e2_v.at[0]], bufs[sub],
                                          sems[sub]).wait()
                    processD(c + sub, cl + sub, sub)
                    issueD(c + sub + 2, sub)
                return _
            lax.fori_loop(0, 16, dpair, None)
            pltpu.sync_copy(
                stage, agg_out.at[pl.ds(wid * 512 + h * 256, 256)])
        pltpu.make_async_copy(ent.at[e2_v.at[0]], bufs[0], sems[0]).wait()
        pltpu.make_async_copy(ent.at[e2_v.at[0]], bufs[1], sems[1]).wait()

    f = pl.kernel(
        body,
        out_type=(jax.ShapeDtypeStruct((1024, 128), jnp.float32),
                  jax.ShapeDtypeStruct((1024, 128), jnp.float32),
                  jax.ShapeDtypeStruct((16384, 128), jnp.float32),
                  jax.ShapeDtypeStruct((16384, 128), jnp.float32),
                  jax.ShapeDtypeStruct((128, 128), jnp.float32)),
        mesh=mesh,
        scratch_types=(
            pltpu.VMEM((1, 32), jnp.int32),
            pltpu.VMEM((1, 32), jnp.int32),
            pltpu.VMEM((32,), jnp.int32),
            pltpu.VMEM((32, 128), jnp.float32),
            pltpu.VMEM((32, 128), jnp.float32),
            pltpu.VMEM((32, 128), jnp.int32),
            pltpu.VMEM((32, 128), jnp.int32),
            pltpu.VMEM((32, 128), jnp.int32),
            pltpu.VMEM((32, 128), jnp.int32),
            pltpu.VMEM((4, 128), jnp.int32),
            pltpu.VMEM((4, 128), jnp.int32),
            pltpu.VMEM((32, 128), jnp.float32),
            pltpu.VMEM((8, 128), jnp.float32),
            pltpu.VMEM((32,), jnp.int32),
            pltpu.VMEM((32,), jnp.int32),
            pltpu.VMEM((64, 128), jnp.int32),
            pltpu.VMEM((64, 128), jnp.int32),
            pltpu.VMEM((128, 128), jnp.float32),
            pltpu.VMEM((128, 128), jnp.float32),
            pltpu.VMEM((256, 128), jnp.float32),
            pltpu.VMEM((4, 128), jnp.float32),
            pltpu.SemaphoreType.DMA,
            pltpu.SemaphoreType.DMA,
            pltpu.SemaphoreType.DMA,
            pltpu.SemaphoreType.DMA,
            pltpu.SemaphoreType.DMA,
            pltpu.SemaphoreType.DMA,
        ),
        compiler_params=pltpu.CompilerParams(needs_layout_passes=False))
    return f(ent_emb, usr_emb, rel_emb, adjE8, adjR8, u32, v32)


def _scores_body(u_ref, rel_ref, out_ref):
    out_ref[...] = lax.dot_general(u_ref[...], rel_ref[...],
                                   (((1,), (1,)), ((), ())),
                                   preferred_element_type=jnp.float32)


def _tc_scores(U, rel_emb):
    B = U.shape[0]
    return pl.pallas_call(
        _scores_body,
        out_shape=jax.ShapeDtypeStruct((B, rel_emb.shape[0]), jnp.float32),
    )(U, rel_emb)


def _dense_body(u_ref, ev0_ref, ev1_ref, ag1_ref, w0_ref, w_ref, b_ref,
                out_ref):
    bb = u_ref.shape[0]
    U = u_ref[...]
    W = w_ref[...]
    bias = b_ref[...]
    EV0 = ev0_ref[...]
    EV1 = ev1_ref[...]
    agg1 = ag1_ref[...]
    w0 = w0_ref[...]

    h1 = jax.nn.sigmoid(
        jnp.dot((EV1 + agg1).reshape(bb * 16, 128), W,
                preferred_element_type=jnp.float32) + bias
    ).reshape(bb, 16, 128)
    agg0 = jnp.sum(w0[..., None] * EV1, axis=1)
    h0 = jax.nn.sigmoid(
        jnp.dot(EV0 + agg0, W, preferred_element_type=jnp.float32) + bias)
    agg0b = jnp.sum(w0[..., None] * h1, axis=1)
    final = jnp.tanh(
        jnp.dot(h0 + agg0b, W, preferred_element_type=jnp.float32) + bias)
    out_ref[...] = jax.nn.sigmoid(jnp.sum(U * final, axis=1))[:, None]


def _tc_dense(U, EV0, EV1, AG1, w0, W, bvec):
    B = U.shape[0]
    bb = 128
    grid = B // bb
    return pl.pallas_call(
        _dense_body,
        grid=(grid,),
        in_specs=[
            pl.BlockSpec((bb, 128), lambda i: (i, 0)),
            pl.BlockSpec((bb, 128), lambda i: (i, 0)),
            pl.BlockSpec((bb, 16, 128), lambda i: (i, 0, 0)),
            pl.BlockSpec((bb, 16, 128), lambda i: (i, 0, 0)),
            pl.BlockSpec((bb, 16), lambda i: (i, 0)),
            pl.BlockSpec((128, 128), lambda i: (0, 0)),
            pl.BlockSpec((1, 128), lambda i: (0, 0)),
        ],
        out_specs=pl.BlockSpec((bb, 1), lambda i: (i, 0)),
        out_shape=jax.ShapeDtypeStruct((B, 1), jnp.float32),
    )(U, EV0, EV1, AG1, w0, W, bvec.reshape(1, 128))


def kernel(ent_emb, usr_emb, rel_emb, W, b, adj_ent, adj_rel, u, v):
    B = u.shape[0]
    n_nb = adj_ent.shape[1]

    U, EV0, EV1, AGG1, W0 = _sc_merged(
        ent_emb, usr_emb, rel_emb,
        adj_ent.reshape(-1, 128), adj_rel.reshape(-1, 128),
        u.reshape(32, 32), v.reshape(32, 32))

    out = _tc_dense(
        U, EV0,
        EV1.reshape(B, n_nb, 128),
        AGG1.reshape(B, n_nb, 128),
        W0.reshape(B, n_nb),
        W, b)
    return out.reshape(B)
